# baseline (speedup 1.0000x reference)
import functools

import jax
import jax.numpy as jnp
from jax import lax
from jax.experimental import pallas as pl
from jax.experimental.pallas import tpu as pltpu
from jax.experimental.pallas import tpu_sc as plsc

N = 4096
F = 512
KP = 1024

_HI = jax.lax.Precision.HIGHEST


def _feat_body(x_ref, w_ref, s_ref, feat_ref, y_ref):
    w = w_ref[...]
    nrm = jax.lax.rsqrt(jnp.maximum(jnp.sum(w * w), 1e-12))
    y = jnp.dot(x_ref[...], w * nrm,
                preferred_element_type=jnp.float32)
    y_ref[...] = y
    feat_ref[:, :F] = x_ref[...] * jnp.tanh(y)
    sbc = lax.bitcast_convert_type(s_ref[...], jnp.float32)
    feat_ref[:, F:] = jnp.broadcast_to(sbc, (N, 128))


def _features(X, w, S):
    return pl.pallas_call(
        _feat_body,
        out_shape=(
            jax.ShapeDtypeStruct((N, F + 128), jnp.float32),
            jax.ShapeDtypeStruct((N, 1), jnp.float32),
        ),
    )(X, w, jnp.reshape(S, (N, 1)))


def _select_body(scol_ref, srow_ref, idx_ref, mask_ref):
    srow = srow_ref[...]
    cw = 512
    for ci in range(N // cw):
        sc = scol_ref[pl.ds(ci * cw, cw), :]
        jj = lax.broadcasted_iota(jnp.int32, (cw, N), 1)
        ii = lax.broadcasted_iota(jnp.int32, (cw, N), 0) + ci * cw
        before = (srow > sc) | ((srow == sc) & (jj < ii))
        rank = jnp.sum(before.astype(jnp.float32), axis=1, keepdims=True)
        mask_ref[pl.ds(ci * cw, cw), :] = (rank < KP).astype(jnp.float32)
    crow = jnp.zeros((1, N), jnp.float32)
    for ci in range(N // cw):
        mc = mask_ref[pl.ds(ci * cw, cw), :]
        jj = lax.broadcasted_iota(jnp.int32, (cw, N), 0) + ci * cw
        ii = lax.broadcasted_iota(jnp.int32, (cw, N), 1)
        crow = crow + jnp.sum(mc * (jj <= ii).astype(jnp.float32),
                              axis=0, keepdims=True)
    for pi in range(KP // cw):
        pp = (lax.broadcasted_iota(jnp.int32, (cw, N), 0) + pi * cw
              ).astype(jnp.float32)
        cnt = jnp.sum((crow <= pp).astype(jnp.float32), axis=1, keepdims=True)
        idx_ref[pl.ds(pi * cw, cw), :] = cnt.astype(jnp.int32)


def _select(y):
    s_col = y
    s_row = jnp.reshape(y, (1, N))
    return pl.pallas_call(
        _select_body,
        out_shape=jax.ShapeDtypeStruct((KP, 1), jnp.int32),
        scratch_shapes=[pltpu.VMEM((N, 1), jnp.float32)],
    )(s_col, s_row)


_TB = 512


def _tr_body(a_ref, o_ref):
    o_ref[...] = a_ref[...].T.astype(jnp.bfloat16)


def _transpose(A):
    g = N // _TB
    return pl.pallas_call(
        _tr_body,
        grid=(g, g),
        in_specs=[pl.BlockSpec((_TB, _TB), lambda i, j: (j, i))],
        out_specs=pl.BlockSpec((_TB, _TB), lambda i, j: (i, j)),
        out_shape=jax.ShapeDtypeStruct((N, N), jnp.bfloat16),
    )(A)


_NC = 2
_NS = 16
_NW = _NC * _NS
_BPW = KP // _NW
_AC = 8


def _sc_gather_a_body(feat_hbm, a_hbm, idx_hbm, idx2_hbm,
                      xp_out, ar_out,
                      idx_v, idxc_v, xbuf, abuf, sem):
    wid = lax.axis_index("s") * _NC + lax.axis_index("c")
    base = wid * _BPW
    pltpu.sync_copy(idx_hbm.at[pl.ds(base, _BPW)], idx_v)
    pltpu.sync_copy(idx2_hbm.at[pl.ds(wid * (_BPW // _AC), _BPW // _AC)],
                    idxc_v)
    pltpu.async_copy(feat_hbm.at[idx_v], xbuf, sem).wait()
    pltpu.sync_copy(xbuf, xp_out.at[pl.ds(base, _BPW)])
    for c in range(_BPW // _AC):
        pltpu.async_copy(a_hbm.at[idxc_v.at[c]], abuf, sem).wait()
        pltpu.sync_copy(abuf, ar_out.at[pl.ds(base + c * _AC, _AC)])


def _sc_gather_a(feat, A, idx, idx2):
    mesh = plsc.VectorSubcoreMesh(core_axis_name="c", subcore_axis_name="s")
    run = functools.partial(
        pl.kernel,
        mesh=mesh,
        out_type=[
            jax.ShapeDtypeStruct((KP, F + 128), jnp.float32),
            jax.ShapeDtypeStruct((KP, N), jnp.float32),
        ],
        scratch_types=[
            pltpu.VMEM((_BPW,), jnp.int32),
            pltpu.VMEM((_BPW // _AC, _AC), jnp.int32),
            pltpu.VMEM((_BPW, F + 128), jnp.float32),
            pltpu.VMEM((_AC, N), jnp.float32),
            pltpu.SemaphoreType.DMA,
        ],
    )(_sc_gather_a_body)
    return run(feat, A, idx, idx2)


def _sc_gather_at_body(at_hbm, idx2_hbm, atr_out, idxc_v, abuf, sem):
    wid = lax.axis_index("s") * _NC + lax.axis_index("c")
    base = wid * _BPW
    pltpu.sync_copy(idx2_hbm.at[pl.ds(wid * (_BPW // _AC), _BPW // _AC)],
                    idxc_v)
    for c in range(_BPW // _AC):
        pltpu.async_copy(at_hbm.at[idxc_v.at[c]], abuf, sem).wait()
        pltpu.sync_copy(abuf, atr_out.at[pl.ds(base + c * _AC, _AC)])


def _sc_gather_at(At, idx2):
    mesh = plsc.VectorSubcoreMesh(core_axis_name="c", subcore_axis_name="s")
    run = functools.partial(
        pl.kernel,
        mesh=mesh,
        out_type=[jax.ShapeDtypeStruct((KP, N // 2), jnp.float32)],
        scratch_types=[
            pltpu.VMEM((_BPW // _AC, _AC), jnp.int32),
            pltpu.VMEM((_AC, N // 2), jnp.float32),
            pltpu.SemaphoreType.DMA,
        ],
    )(_sc_gather_at_body)
    return run(At, idx2)[0]


_MB = 512


def _mm_body(ar_ref, atr_ref, o_ref):
    o_ref[...] = lax.dot_general(
        ar_ref[...].astype(jnp.bfloat16), atr_ref[...],
        (((1,), (1,)), ((), ())),
        preferred_element_type=jnp.float32)


def _pool_matmul(Ar, Atr):
    g = KP // _MB
    return pl.pallas_call(
        _mm_body,
        grid=(g, g),
        in_specs=[
            pl.BlockSpec((_MB, N), lambda i, j: (i, 0)),
            pl.BlockSpec((_MB, N), lambda i, j: (j, 0)),
        ],
        out_specs=pl.BlockSpec((_MB, _MB), lambda i, j: (i, j)),
        out_shape=jax.ShapeDtypeStruct((KP, KP), jnp.float32),
    )(Ar, Atr)


def kernel(X, A, S, kernel):
    feat, y = _features(X, kernel, S)
    idx2d = _select(y)
    idx = jnp.reshape(idx2d, (KP,))
    idx2 = jnp.reshape(idx, (KP // _AC, _AC))
    G, Ar = _sc_gather_a(feat, A, idx, idx2)
    At = _transpose(A)
    At32 = lax.bitcast_convert_type(jnp.reshape(At, (N, N // 2, 2)),
                                    jnp.float32)
    Atr32 = _sc_gather_at(At32, idx2)
    Atr = jnp.reshape(lax.bitcast_convert_type(Atr32, jnp.bfloat16),
                      (KP, N))
    Ap = _pool_matmul(Ar, Atr)
    Xp = G[:, :F]
    Sp = lax.bitcast_convert_type(G[:, F], jnp.int32)
    return Xp, Ap, Sp


# speedup vs baseline: 3.0767x; 3.0767x over previous
import functools

import jax
import jax.numpy as jnp
from jax import lax
from jax.experimental import pallas as pl
from jax.experimental.pallas import tpu as pltpu
from jax.experimental.pallas import tpu_sc as plsc

N = 4096
F = 512
KP = 1024

_HI = jax.lax.Precision.HIGHEST


def _feat_body(x_ref, w_ref, s_ref, feat_ref, y_ref):
    w = w_ref[...]
    nrm = jax.lax.rsqrt(jnp.maximum(jnp.sum(w * w), 1e-12))
    y = jnp.dot(x_ref[...], w * nrm,
                preferred_element_type=jnp.float32)
    y_ref[...] = y
    feat_ref[:, :F] = x_ref[...] * jnp.tanh(y)
    sbc = lax.bitcast_convert_type(s_ref[...], jnp.float32)
    feat_ref[:, F:] = jnp.broadcast_to(sbc, (N, 128))


def _features(X, w, S):
    return pl.pallas_call(
        _feat_body,
        out_shape=(
            jax.ShapeDtypeStruct((N, F + 128), jnp.float32),
            jax.ShapeDtypeStruct((N, 1), jnp.float32),
        ),
    )(X, w, jnp.reshape(S, (N, 1)))


def _select_body(scol_ref, srow_ref, idx_ref, mask_ref):
    srow = srow_ref[...]
    cw = 512
    for ci in range(N // cw):
        sc = scol_ref[pl.ds(ci * cw, cw), :]
        jj = lax.broadcasted_iota(jnp.int32, (cw, N), 1)
        ii = lax.broadcasted_iota(jnp.int32, (cw, N), 0) + ci * cw
        before = (srow > sc) | ((srow == sc) & (jj < ii))
        rank = jnp.sum(before.astype(jnp.float32), axis=1, keepdims=True)
        mask_ref[pl.ds(ci * cw, cw), :] = (rank < KP).astype(jnp.float32)
    crow = jnp.zeros((1, N), jnp.float32)
    for ci in range(N // cw):
        mc = mask_ref[pl.ds(ci * cw, cw), :]
        jj = lax.broadcasted_iota(jnp.int32, (cw, N), 0) + ci * cw
        ii = lax.broadcasted_iota(jnp.int32, (cw, N), 1)
        crow = crow + jnp.sum(mc * (jj <= ii).astype(jnp.float32),
                              axis=0, keepdims=True)
    for pi in range(KP // cw):
        pp = (lax.broadcasted_iota(jnp.int32, (cw, N), 0) + pi * cw
              ).astype(jnp.float32)
        cnt = jnp.sum((crow <= pp).astype(jnp.float32), axis=1, keepdims=True)
        idx_ref[pl.ds(pi * cw, cw), :] = cnt.astype(jnp.int32)


def _select(y):
    s_col = y
    s_row = jnp.reshape(y, (1, N))
    return pl.pallas_call(
        _select_body,
        out_shape=jax.ShapeDtypeStruct((KP, 1), jnp.int32),
        scratch_shapes=[pltpu.VMEM((N, 1), jnp.float32)],
    )(s_col, s_row)


_TB = 512


def _tr_body(a_ref, o_ref):
    o_ref[...] = a_ref[...].T


def _transpose(A):
    g = N // _TB
    return pl.pallas_call(
        _tr_body,
        grid=(g, g),
        in_specs=[pl.BlockSpec((_TB, _TB), lambda i, j: (j, i))],
        out_specs=pl.BlockSpec((_TB, _TB), lambda i, j: (i, j)),
        out_shape=jax.ShapeDtypeStruct((N, N), jnp.float32),
    )(A)


_NC = 2
_NS = 16
_NW = _NC * _NS
_BPW = KP // _NW
_AC = 8


def _sc_gather_a_body(feat_hbm, a_hbm, idx_hbm, idx2_hbm,
                      xp_out, ar_out,
                      idx_v, idxc_v, xbuf, abuf, sem):
    wid = lax.axis_index("s") * _NC + lax.axis_index("c")
    base = wid * _BPW
    pltpu.sync_copy(idx_hbm.at[pl.ds(base, _BPW)], idx_v)
    pltpu.sync_copy(idx2_hbm.at[pl.ds(wid * (_BPW // _AC), _BPW // _AC)],
                    idxc_v)
    pltpu.async_copy(feat_hbm.at[idx_v], xbuf, sem).wait()
    pltpu.sync_copy(xbuf, xp_out.at[pl.ds(base, _BPW)])
    for c in range(_BPW // _AC):
        pltpu.async_copy(a_hbm.at[idxc_v.at[c]], abuf, sem).wait()
        pltpu.sync_copy(abuf, ar_out.at[pl.ds(base + c * _AC, _AC)])


def _sc_gather_a(feat, A, idx, idx2):
    mesh = plsc.VectorSubcoreMesh(core_axis_name="c", subcore_axis_name="s")
    run = functools.partial(
        pl.kernel,
        mesh=mesh,
        out_type=[
            jax.ShapeDtypeStruct((KP, F + 128), jnp.float32),
            jax.ShapeDtypeStruct((KP, N), jnp.float32),
        ],
        scratch_types=[
            pltpu.VMEM((_BPW,), jnp.int32),
            pltpu.VMEM((_BPW // _AC, _AC), jnp.int32),
            pltpu.VMEM((_BPW, F + 128), jnp.float32),
            pltpu.VMEM((_AC, N), jnp.float32),
            pltpu.SemaphoreType.DMA,
        ],
    )(_sc_gather_a_body)
    return run(feat, A, idx, idx2)


def _sc_gather_at_body(at_hbm, idx2_hbm, atr_out, idxc_v, abuf, sem):
    wid = lax.axis_index("s") * _NC + lax.axis_index("c")
    base = wid * _BPW
    pltpu.sync_copy(idx2_hbm.at[pl.ds(wid * (_BPW // _AC), _BPW // _AC)],
                    idxc_v)
    for c in range(_BPW // _AC):
        pltpu.async_copy(at_hbm.at[idxc_v.at[c]], abuf, sem).wait()
        pltpu.sync_copy(abuf, atr_out.at[pl.ds(base + c * _AC, _AC)])


def _sc_gather_at(At, idx2):
    mesh = plsc.VectorSubcoreMesh(core_axis_name="c", subcore_axis_name="s")
    run = functools.partial(
        pl.kernel,
        mesh=mesh,
        out_type=[jax.ShapeDtypeStruct((KP, N), jnp.float32)],
        scratch_types=[
            pltpu.VMEM((_BPW // _AC, _AC), jnp.int32),
            pltpu.VMEM((_AC, N), jnp.float32),
            pltpu.SemaphoreType.DMA,
        ],
    )(_sc_gather_at_body)
    return run(At, idx2)[0]


_MB = 512


def _mm_body(ar_ref, atr_ref, o_ref):
    o_ref[...] = lax.dot_general(
        ar_ref[...], atr_ref[...], (((1,), (1,)), ((), ())),
        preferred_element_type=jnp.float32)


def _pool_matmul(Ar, Atr):
    g = KP // _MB
    return pl.pallas_call(
        _mm_body,
        grid=(g, g),
        in_specs=[
            pl.BlockSpec((_MB, N), lambda i, j: (i, 0)),
            pl.BlockSpec((_MB, N), lambda i, j: (j, 0)),
        ],
        out_specs=pl.BlockSpec((_MB, _MB), lambda i, j: (i, j)),
        out_shape=jax.ShapeDtypeStruct((KP, KP), jnp.float32),
    )(Ar, Atr)


def kernel(X, A, S, kernel):
    feat, y = _features(X, kernel, S)
    idx2d = _select(y)
    idx = jnp.reshape(idx2d, (KP,))
    idx2 = jnp.reshape(idx, (KP // _AC, _AC))
    G, Ar = _sc_gather_a(feat, A, idx, idx2)
    At = _transpose(A)
    Atr = _sc_gather_at(At, idx2)
    Ap = _pool_matmul(Ar, Atr)
    Xp = G[:, :F]
    Sp = lax.bitcast_convert_type(G[:, F], jnp.int32)
    return Xp, Ap, Sp


# speedup vs baseline: 3.3187x; 1.0787x over previous
import functools

import jax
import jax.numpy as jnp
from jax import lax
from jax.experimental import pallas as pl
from jax.experimental.pallas import tpu as pltpu
from jax.experimental.pallas import tpu_sc as plsc

N = 4096
F = 512
KP = 1024


def _head_body(x_ref, w_ref, s_ref, feat_ref, idx_ref, mask_ref):
    w = w_ref[...]
    nrm = jax.lax.rsqrt(jnp.maximum(jnp.sum(w * w), 1e-12))
    y = jnp.dot(x_ref[...], w * nrm,
                preferred_element_type=jnp.float32)
    feat_ref[:, :F] = x_ref[...] * jnp.tanh(y)
    sbc = lax.bitcast_convert_type(s_ref[...], jnp.float32)
    feat_ref[:, F:] = jnp.broadcast_to(sbc, (N, 128))

    srow = y.T
    cw = 512
    for ci in range(N // cw):
        sc = y[ci * cw:(ci + 1) * cw, :]
        jj = lax.broadcasted_iota(jnp.int32, (cw, N), 1)
        ii = lax.broadcasted_iota(jnp.int32, (cw, N), 0) + ci * cw
        before = (srow > sc) | ((srow == sc) & (jj < ii))
        rank = jnp.sum(before.astype(jnp.float32), axis=1, keepdims=True)
        mask_ref[pl.ds(ci * cw, cw), :] = (rank < KP).astype(jnp.float32)
    crow = jnp.zeros((1, N), jnp.float32)
    for ci in range(N // cw):
        mc = mask_ref[pl.ds(ci * cw, cw), :]
        jj = lax.broadcasted_iota(jnp.int32, (cw, N), 0) + ci * cw
        ii = lax.broadcasted_iota(jnp.int32, (cw, N), 1)
        crow = crow + jnp.sum(mc * (jj <= ii).astype(jnp.float32),
                              axis=0, keepdims=True)
    for pi in range(KP // cw):
        pp = (lax.broadcasted_iota(jnp.int32, (cw, N), 0) + pi * cw
              ).astype(jnp.float32)
        cnt = jnp.sum((crow <= pp).astype(jnp.float32), axis=1, keepdims=True)
        idx_ref[pl.ds(pi * cw, cw), :] = cnt.astype(jnp.int32)


def _head(X, w, S):
    return pl.pallas_call(
        _head_body,
        out_shape=(
            jax.ShapeDtypeStruct((N, F + 128), jnp.float32),
            jax.ShapeDtypeStruct((KP, 1), jnp.int32),
        ),
        scratch_shapes=[pltpu.VMEM((N, 1), jnp.float32)],
    )(X, w, jnp.reshape(S, (N, 1)))


_TB = 512


def _tr_body(a_ref, o_ref):
    o_ref[...] = a_ref[...].T


def _transpose(A):
    g = N // _TB
    return pl.pallas_call(
        _tr_body,
        grid=(g, g),
        in_specs=[pl.BlockSpec((_TB, _TB), lambda i, j: (j, i))],
        out_specs=pl.BlockSpec((_TB, _TB), lambda i, j: (i, j)),
        out_shape=jax.ShapeDtypeStruct((N, N), jnp.float32),
    )(A)


_NC = 2
_NS = 16
_NW = _NC * _NS
_BPW = KP // _NW
_AC = 8


def _sc_gather_a_body(feat_hbm, a_hbm, idx_hbm, idx2_hbm,
                      xp_out, sp_out, ar_out,
                      idx_v, idxc_v, xbuf, abuf, sem):
    wid = lax.axis_index("s") * _NC + lax.axis_index("c")
    base = wid * _BPW
    pltpu.sync_copy(idx_hbm.at[pl.ds(base, _BPW)], idx_v)
    pltpu.sync_copy(idx2_hbm.at[pl.ds(wid * (_BPW // _AC), _BPW // _AC)],
                    idxc_v)
    pltpu.async_copy(feat_hbm.at[idx_v], xbuf, sem).wait()
    pltpu.sync_copy(xbuf.at[:, :F], xp_out.at[pl.ds(base, _BPW)])
    pltpu.sync_copy(xbuf.at[:, F:], sp_out.at[pl.ds(base, _BPW)])
    for c in range(_BPW // _AC):
        pltpu.async_copy(a_hbm.at[idxc_v.at[c]], abuf, sem).wait()
        pltpu.sync_copy(abuf, ar_out.at[pl.ds(base + c * _AC, _AC)])


def _sc_gather_a(feat, A, idx, idx2):
    mesh = plsc.VectorSubcoreMesh(core_axis_name="c", subcore_axis_name="s")
    run = functools.partial(
        pl.kernel,
        mesh=mesh,
        out_type=[
            jax.ShapeDtypeStruct((KP, F), jnp.float32),
            jax.ShapeDtypeStruct((KP, 128), jnp.float32),
            jax.ShapeDtypeStruct((KP, N), jnp.float32),
        ],
        scratch_types=[
            pltpu.VMEM((_BPW,), jnp.int32),
            pltpu.VMEM((_BPW // _AC, _AC), jnp.int32),
            pltpu.VMEM((_BPW, F + 128), jnp.float32),
            pltpu.VMEM((_AC, N), jnp.float32),
            pltpu.SemaphoreType.DMA,
        ],
    )(_sc_gather_a_body)
    return run(feat, A, idx, idx2)


def _sc_gather_at_body(at_hbm, idx2_hbm, atr_out, idxc_v, abuf, sem):
    wid = lax.axis_index("s") * _NC + lax.axis_index("c")
    base = wid * _BPW
    pltpu.sync_copy(idx2_hbm.at[pl.ds(wid * (_BPW // _AC), _BPW // _AC)],
                    idxc_v)
    for c in range(_BPW // _AC):
        pltpu.async_copy(at_hbm.at[idxc_v.at[c]], abuf, sem).wait()
        pltpu.sync_copy(abuf, atr_out.at[pl.ds(base + c * _AC, _AC)])


def _sc_gather_at(At, idx2):
    mesh = plsc.VectorSubcoreMesh(core_axis_name="c", subcore_axis_name="s")
    run = functools.partial(
        pl.kernel,
        mesh=mesh,
        out_type=[jax.ShapeDtypeStruct((KP, N), jnp.float32)],
        scratch_types=[
            pltpu.VMEM((_BPW // _AC, _AC), jnp.int32),
            pltpu.VMEM((_AC, N), jnp.float32),
            pltpu.SemaphoreType.DMA,
        ],
    )(_sc_gather_at_body)
    return run(At, idx2)[0]


_MB = 256


def _mm_body(ar_ref, atr_ref, o_ref):
    o_ref[...] = lax.dot_general(
        ar_ref[...], atr_ref[...], (((1,), (1,)), ((), ())),
        preferred_element_type=jnp.float32)


def _pool_matmul(Ar, Atr):
    g = KP // _MB
    return pl.pallas_call(
        _mm_body,
        grid=(g,),
        in_specs=[
            pl.BlockSpec((_MB, N), lambda i: (i, 0)),
            pl.BlockSpec((KP, N), lambda i: (0, 0)),
        ],
        out_specs=pl.BlockSpec((_MB, KP), lambda i: (i, 0)),
        out_shape=jax.ShapeDtypeStruct((KP, KP), jnp.float32),
    )(Ar, Atr)


def kernel(X, A, S, kernel):
    feat, idx2d = _head(X, kernel, S)
    idx = jnp.reshape(idx2d, (KP,))
    idx2 = jnp.reshape(idx, (KP // _AC, _AC))
    Xp, Sp2, Ar = _sc_gather_a(feat, A, idx, idx2)
    At = _transpose(A)
    Atr = _sc_gather_at(At, idx2)
    Ap = _pool_matmul(Ar, Atr)
    Sp = lax.bitcast_convert_type(Sp2[:, 0], jnp.int32)
    return Xp, Ap, Sp


# speedup vs baseline: 4.4927x; 1.3538x over previous
import functools

import jax
import jax.numpy as jnp
from jax import lax
from jax.experimental import pallas as pl
from jax.experimental.pallas import tpu as pltpu
from jax.experimental.pallas import tpu_sc as plsc

N = 4096
F = 512
KP = 1024


def _head_body(x_ref, w_ref, s_ref, feat_ref, idx_ref, mask_ref):
    w = w_ref[...]
    nrm = jax.lax.rsqrt(jnp.maximum(jnp.sum(w * w), 1e-12))
    y = jnp.dot(x_ref[...], w * nrm,
                preferred_element_type=jnp.float32)
    feat_ref[:, :F] = x_ref[...] * jnp.tanh(y)
    sbc = lax.bitcast_convert_type(s_ref[...], jnp.float32)
    feat_ref[:, F:] = jnp.broadcast_to(sbc, (N, 128))

    srow = y.T
    cw = 512
    for ci in range(N // cw):
        sc = y[ci * cw:(ci + 1) * cw, :]
        jj = lax.broadcasted_iota(jnp.int32, (cw, N), 1)
        ii = lax.broadcasted_iota(jnp.int32, (cw, N), 0) + ci * cw
        before = (srow > sc) | ((srow == sc) & (jj < ii))
        rank = jnp.sum(before.astype(jnp.float32), axis=1, keepdims=True)
        mask_ref[pl.ds(ci * cw, cw), :] = (rank < KP).astype(jnp.float32)
    crow = jnp.zeros((1, N), jnp.float32)
    for ci in range(N // cw):
        mc = mask_ref[pl.ds(ci * cw, cw), :]
        jj = lax.broadcasted_iota(jnp.int32, (cw, N), 0) + ci * cw
        ii = lax.broadcasted_iota(jnp.int32, (cw, N), 1)
        crow = crow + jnp.sum(mc * (jj <= ii).astype(jnp.float32),
                              axis=0, keepdims=True)
    for pi in range(KP // cw):
        pp = (lax.broadcasted_iota(jnp.int32, (cw, N), 0) + pi * cw
              ).astype(jnp.float32)
        cnt = jnp.sum((crow <= pp).astype(jnp.float32), axis=1, keepdims=True)
        idx_ref[pl.ds(pi * cw, cw), :] = cnt.astype(jnp.int32)


def _head(X, w, S):
    return pl.pallas_call(
        _head_body,
        out_shape=(
            jax.ShapeDtypeStruct((N, F + 128), jnp.float32),
            jax.ShapeDtypeStruct((KP, 1), jnp.int32),
        ),
        scratch_shapes=[pltpu.VMEM((N, 1), jnp.float32)],
    )(X, w, jnp.reshape(S, (N, 1)))


_TRG = 4
_TBJ = N // _TRG


def _tr_body(a1_ref, a2_ref, o_ref):
    lo = a1_ref[...].T.astype(jnp.bfloat16)
    hi = a2_ref[...].T.astype(jnp.bfloat16)
    lo32 = lax.convert_element_type(
        lax.bitcast_convert_type(lo, jnp.uint16), jnp.uint32)
    hi32 = lax.convert_element_type(
        lax.bitcast_convert_type(hi, jnp.uint16), jnp.uint32)
    o_ref[...] = ((hi32 << 16) | lo32).astype(jnp.int32)


def _transpose_packed(A):
    kb = N // 2 // _TRG
    return pl.pallas_call(
        _tr_body,
        grid=(_TRG, _TRG),
        in_specs=[
            pl.BlockSpec((kb, _TBJ), lambda g, j: (g, j)),
            pl.BlockSpec((kb, _TBJ), lambda g, j: (g + _TRG, j)),
        ],
        out_specs=pl.BlockSpec((_TBJ, kb), lambda g, j: (j, g)),
        out_shape=jax.ShapeDtypeStruct((N, N // 2), jnp.int32),
    )(A, A)


_NC = 2
_NS = 16
_NW = _NC * _NS
_BPW = KP // _NW
_AC = 8


def _sc_gather_a_body(feat_hbm, a_hbm, idx_hbm, idx2_hbm,
                      xp_out, sp_out, ar_out,
                      idx_v, idxc_v, xbuf, abuf, sem):
    wid = lax.axis_index("s") * _NC + lax.axis_index("c")
    base = wid * _BPW
    pltpu.sync_copy(idx_hbm.at[pl.ds(base, _BPW)], idx_v)
    pltpu.sync_copy(idx2_hbm.at[pl.ds(wid * (_BPW // _AC), _BPW // _AC)],
                    idxc_v)
    pltpu.async_copy(feat_hbm.at[idx_v], xbuf, sem).wait()
    pltpu.sync_copy(xbuf.at[:, :F], xp_out.at[pl.ds(base, _BPW)])
    pltpu.sync_copy(xbuf.at[:, F:], sp_out.at[pl.ds(base, _BPW)])
    for c in range(_BPW // _AC):
        pltpu.async_copy(a_hbm.at[idxc_v.at[c]], abuf, sem).wait()
        pltpu.sync_copy(abuf, ar_out.at[pl.ds(base + c * _AC, _AC)])


def _sc_gather_a(feat, A, idx, idx2):
    mesh = plsc.VectorSubcoreMesh(core_axis_name="c", subcore_axis_name="s")
    run = functools.partial(
        pl.kernel,
        mesh=mesh,
        out_type=[
            jax.ShapeDtypeStruct((KP, F), jnp.float32),
            jax.ShapeDtypeStruct((KP, 128), jnp.float32),
            jax.ShapeDtypeStruct((KP, N), jnp.float32),
        ],
        scratch_types=[
            pltpu.VMEM((_BPW,), jnp.int32),
            pltpu.VMEM((_BPW // _AC, _AC), jnp.int32),
            pltpu.VMEM((_BPW, F + 128), jnp.float32),
            pltpu.VMEM((_AC, N), jnp.float32),
            pltpu.SemaphoreType.DMA,
        ],
    )(_sc_gather_a_body)
    return run(feat, A, idx, idx2)


def _sc_gather_at_body(at_hbm, idx2_hbm, atr_out, idxc_v, abuf, sem):
    wid = lax.axis_index("s") * _NC + lax.axis_index("c")
    base = wid * _BPW
    pltpu.sync_copy(idx2_hbm.at[pl.ds(wid * (_BPW // _AC), _BPW // _AC)],
                    idxc_v)
    for c in range(_BPW // _AC):
        pltpu.async_copy(at_hbm.at[idxc_v.at[c]], abuf, sem).wait()
        pltpu.sync_copy(abuf, atr_out.at[pl.ds(base + c * _AC, _AC)])


def _sc_gather_at(At, idx2):
    mesh = plsc.VectorSubcoreMesh(core_axis_name="c", subcore_axis_name="s")
    run = functools.partial(
        pl.kernel,
        mesh=mesh,
        out_type=[jax.ShapeDtypeStruct((KP, N // 2), jnp.int32)],
        scratch_types=[
            pltpu.VMEM((_BPW // _AC, _AC), jnp.int32),
            pltpu.VMEM((_AC, N // 2), jnp.int32),
            pltpu.SemaphoreType.DMA,
        ],
    )(_sc_gather_at_body)
    return run(At, idx2)[0]


_MB = 256


def _mm_body(ar_ref, atrp_ref, o_ref):
    w = lax.bitcast_convert_type(atrp_ref[...], jnp.uint32)
    lo = lax.bitcast_convert_type(
        lax.convert_element_type(w & 0xFFFF, jnp.uint16), jnp.bfloat16)
    hi = lax.bitcast_convert_type(
        lax.convert_element_type(w >> 16, jnp.uint16), jnp.bfloat16)
    ar = ar_ref[...]
    dn = (((1,), (1,)), ((), ()))
    o_ref[...] = (
        lax.dot_general(ar[:, :N // 2].astype(jnp.bfloat16), lo, dn,
                        preferred_element_type=jnp.float32)
        + lax.dot_general(ar[:, N // 2:].astype(jnp.bfloat16), hi, dn,
                          preferred_element_type=jnp.float32))


def _pool_matmul(Ar, Atrp):
    g = KP // _MB
    return pl.pallas_call(
        _mm_body,
        grid=(g,),
        in_specs=[
            pl.BlockSpec((_MB, N), lambda i: (i, 0)),
            pl.BlockSpec((KP, N // 2), lambda i: (0, 0)),
        ],
        out_specs=pl.BlockSpec((_MB, KP), lambda i: (i, 0)),
        out_shape=jax.ShapeDtypeStruct((KP, KP), jnp.float32),
    )(Ar, Atrp)


def kernel(X, A, S, kernel):
    feat, idx2d = _head(X, kernel, S)
    idx = jnp.reshape(idx2d, (KP,))
    idx2 = jnp.reshape(idx, (KP // _AC, _AC))
    Xp, Sp2, Ar = _sc_gather_a(feat, A, idx, idx2)
    At = _transpose_packed(A)
    Atr = _sc_gather_at(At, idx2)
    Ap = _pool_matmul(Ar, Atr)
    Sp = lax.bitcast_convert_type(Sp2[:, 0], jnp.int32)
    return Xp, Ap, Sp
